# baseline (speedup 1.0000x reference)
import jax
import jax.numpy as jnp
from jax import lax
from jax.experimental import pallas as pl

NUM_CLASSES = 1000
BLOCK_ROWS = 64
BLOCK_C = 128


def _onehot_body(x_ref, out_ref):
    j = pl.program_id(1)
    idx = x_ref[...]
    classes = lax.broadcasted_iota(
        jnp.int32, (BLOCK_ROWS, 20, BLOCK_C), 2) + j * BLOCK_C
    out_ref[...] = (idx[:, :, None] == classes).astype(jnp.float32)


def kernel(x):
    B, S = x.shape
    grid = (B // BLOCK_ROWS, pl.cdiv(NUM_CLASSES, BLOCK_C))
    return pl.pallas_call(
        _onehot_body,
        grid=grid,
        in_specs=[pl.BlockSpec((BLOCK_ROWS, S), lambda i, j: (i, 0))],
        out_specs=pl.BlockSpec((BLOCK_ROWS, S, BLOCK_C), lambda i, j: (i, 0, j)),
        out_shape=jax.ShapeDtypeStruct((B, S, NUM_CLASSES), jnp.float32),
    )(x.astype(jnp.int32))


# speedup vs baseline: 1.7333x; 1.7333x over previous
import jax
import jax.numpy as jnp
from jax import lax
from jax.experimental import pallas as pl

NUM_CLASSES = 1000
S_PAD = 24
C_PAD = 1024
BLOCK_ROWS = 128


def _onehot_body(x_ref, out_ref):
    idx = x_ref[...]
    idx = jnp.concatenate(
        [idx, jnp.full((BLOCK_ROWS, S_PAD - 20), -1, jnp.int32)], axis=1)
    classes = lax.broadcasted_iota(jnp.int32, (BLOCK_ROWS, S_PAD, C_PAD), 2)
    out_ref[...] = (idx[:, :, None] == classes).astype(jnp.float32)


def kernel(x):
    B, S = x.shape
    grid = (B // BLOCK_ROWS,)
    padded = pl.pallas_call(
        _onehot_body,
        grid=grid,
        in_specs=[pl.BlockSpec((BLOCK_ROWS, S), lambda i: (i, 0))],
        out_specs=pl.BlockSpec((BLOCK_ROWS, S_PAD, C_PAD), lambda i: (i, 0, 0)),
        out_shape=jax.ShapeDtypeStruct((B, S_PAD, C_PAD), jnp.float32),
    )(x.astype(jnp.int32))
    return padded[:, :S, :NUM_CLASSES]
